# baseline (speedup 1.0000x reference)
import functools

import jax
import jax.numpy as jnp
import numpy as np
from jax import lax
from jax.experimental import pallas as pl
from jax.experimental.pallas import tpu as pltpu
from jax.experimental.pallas import tpu_sc as plsc

N = 10000
ROWS = 10240
D_IN = 128
D = 64
ACC_W = 80
E_RAW = 320000
E_TOT = E_RAW + N
K = 128
NC, NS = 2, 16
NW = NC * NS
CPW = 2 * (-(-E_TOT // (2 * K * NW)))
G = CPW * NW
E_PAD = G * K
RPT = ROWS // NS
NCOPY = RPT // K


def _edge_body(H, a_hbm, b_hbm, src_hbm, dst_hbm, att_hbm, out_hbm,
               idx_all_s, idx_all_d, arows, brows, stage, att_v, acc,
               sem_a, sem_b, sem_s):
    cid = lax.axis_index("c")
    sid = lax.axis_index("s")
    cph = D // H

    pltpu.sync_copy(att_hbm, att_v)

    zero = jnp.zeros((16,), jnp.float32)

    def zrow(i, c):
        for p in range(2):
            for j in range(ACC_W // 16):
                stage[p, i, pl.ds(j * 16, 16)] = zero
        return c

    lax.fori_loop(0, K, zrow, 0)
    for j in range(NCOPY):
        pltpu.sync_copy(stage.at[0], acc.at[pl.ds(sid * RPT + j * K, K)])
    plsc.subcore_barrier()

    wid = sid * NC + cid
    lane = lax.iota(jnp.int32, 16)

    pltpu.sync_copy(src_hbm.at[pl.ds(wid * CPW, CPW)], idx_all_s)
    pltpu.sync_copy(dst_hbm.at[pl.ds(wid * CPW, CPW)], idx_all_d)

    def gather_pair(p, t):
        pltpu.async_copy(a_hbm.at[idx_all_s.at[t]], arows.at[p], sem_a.at[p])
        pltpu.async_copy(b_hbm.at[idx_all_d.at[t]], brows.at[p], sem_b.at[p])

    def compute_chunk(p):
        def group_body(gg, c2):
            rows = lane + gg * 16
            cols = [(ch // cph) * cph + ((ch % cph + lane) & (cph - 1))
                    for ch in range(D)]
            scores = [jnp.zeros((16,), jnp.float32) for _ in range(H)]
            for ch in range(D):
                av = plsc.load_gather(arows.at[p], [rows, cols[ch]])
                bv = plsc.load_gather(brows.at[p], [rows, cols[ch]])
                v = av + bv
                e = jnp.maximum(v, 0.2 * v)
                scores[ch // cph] = scores[ch // cph] + att_v[ch] * e
            exs = []
            for h in range(H):
                exh = jnp.exp(scores[h])
                exs.append(exh)
                plsc.store_scatter(stage.at[p], [rows, jnp.full((16,), D + h, jnp.int32)], exh)
            for ch in range(D):
                av = plsc.load_gather(arows.at[p], [rows, cols[ch]])
                plsc.store_scatter(stage.at[p], [rows, cols[ch]], exs[ch // cph] * av)
            return c2

        lax.fori_loop(0, K // 16, group_body, 0)

    pltpu.async_copy(stage.at[1], acc.at[idx_all_d.at[0]], sem_s.at[1], add=True)
    gather_pair(0, 0)

    def pipe_body(t0, c):
        for b in range(2):
            t = t0 + b
            p = b
            q = 1 - b
            pltpu.make_async_copy(stage.at[q], acc.at[idx_all_d.at[t]], sem_s.at[q]).wait()

            @pl.when(t + 1 < CPW)
            def _():
                gather_pair(q, t + 1)

            pltpu.make_async_copy(a_hbm.at[idx_all_s.at[t]], arows.at[p], sem_a.at[p]).wait()
            pltpu.make_async_copy(b_hbm.at[idx_all_d.at[t]], brows.at[p], sem_b.at[p]).wait()
            compute_chunk(p)
            pltpu.async_copy(stage.at[p], acc.at[idx_all_d.at[t]], sem_s.at[p], add=True)
        return c

    lax.fori_loop(0, CPW // 2, lambda i, c: pipe_body(i * 2, c), 0)
    pltpu.make_async_copy(stage.at[1], acc.at[idx_all_d.at[0]], sem_s.at[1]).wait()
    plsc.subcore_barrier()
    for j in range(NCOPY):
        sl = pl.ds(sid * RPT + j * K, K)
        pltpu.sync_copy(acc.at[sl], out_hbm.at[cid, sl])


def _make_edge_pass(H):
    body = functools.partial(_edge_body, H)
    return pl.kernel(
        body,
        out_type=jax.ShapeDtypeStruct((NC, ROWS, ACC_W), jnp.float32),
        mesh=plsc.VectorSubcoreMesh(core_axis_name="c", subcore_axis_name="s"),
        scratch_types=[
            pltpu.VMEM((CPW, K), jnp.int32),
            pltpu.VMEM((CPW, K), jnp.int32),
            pltpu.VMEM((2, K, D), jnp.float32),
            pltpu.VMEM((2, K, D), jnp.float32),
            pltpu.VMEM((2, K, ACC_W), jnp.float32),
            pltpu.VMEM((D, 16), jnp.float32),
            pltpu.VMEM_SHARED((ROWS, ACC_W), jnp.float32),
            pltpu.SemaphoreType.DMA((2,)),
            pltpu.SemaphoreType.DMA((2,)),
            pltpu.SemaphoreType.DMA((2,)),
        ],
        compiler_params=pltpu.CompilerParams(needs_layout_passes=False,
                                             use_tc_tiling_on_sc=False),
    )


_edge_pass_h4 = _make_edge_pass(4)
_edge_pass_h2 = _make_edge_pass(2)


def _pre_body(x_ref, wl_ref, bl_ref, wr_ref, br_ref, xl_ref, xr_ref):
    xv = x_ref[...]
    xl_ref[...] = jnp.dot(xv, wl_ref[...], preferred_element_type=jnp.float32) + bl_ref[...]
    xr_ref[...] = jnp.dot(xv, wr_ref[...], preferred_element_type=jnp.float32) + br_ref[...]


@jax.jit
def _pre(x, wl, bl, wr, br):
    blk = 1024
    grid = ROWS // blk
    return pl.pallas_call(
        _pre_body,
        grid=(grid,),
        in_specs=[
            pl.BlockSpec((blk, D_IN), lambda i: (i, 0)),
            pl.BlockSpec((D_IN, D), lambda i: (0, 0)),
            pl.BlockSpec((1, D), lambda i: (0, 0)),
            pl.BlockSpec((D_IN, D), lambda i: (0, 0)),
            pl.BlockSpec((1, D), lambda i: (0, 0)),
        ],
        out_specs=[
            pl.BlockSpec((blk, D), lambda i: (i, 0)),
            pl.BlockSpec((blk, D), lambda i: (i, 0)),
        ],
        out_shape=[
            jax.ShapeDtypeStruct((ROWS, D), jnp.float32),
            jax.ShapeDtypeStruct((ROWS, D), jnp.float32),
        ],
    )(x, wl, bl, wr, br)


def _mid_body(p_ref, sel_ref, b1_ref, wl_ref, bl_ref, wr_ref, br_ref, a_ref, b_ref):
    p0 = p_ref[0]
    p1 = p_ref[1]
    num = p0[:, :D] + p1[:, :D]
    den4 = p0[:, D:D + 4] + p1[:, D:D + 4]
    den = jnp.dot(den4, sel_ref[...], preferred_element_type=jnp.float32)
    h = jnp.maximum(num / (den + 1e-16) + b1_ref[...], 0.0)
    a_ref[...] = jnp.dot(h, wl_ref[...], preferred_element_type=jnp.float32) + bl_ref[...]
    b_ref[...] = jnp.dot(h, wr_ref[...], preferred_element_type=jnp.float32) + br_ref[...]


@jax.jit
def _mid(p, b1, wl23, bl23, wr23, br23):
    blk = 1024
    grid = ROWS // blk
    sel = jnp.asarray(np.repeat(np.eye(4, dtype=np.float32), 16, axis=1))
    return pl.pallas_call(
        _mid_body,
        grid=(grid,),
        in_specs=[
            pl.BlockSpec((NC, blk, ACC_W), lambda i: (0, i, 0)),
            pl.BlockSpec((4, D), lambda i: (0, 0)),
            pl.BlockSpec((1, D), lambda i: (0, 0)),
            pl.BlockSpec((D, D), lambda i: (0, 0)),
            pl.BlockSpec((1, D), lambda i: (0, 0)),
            pl.BlockSpec((D, D), lambda i: (0, 0)),
            pl.BlockSpec((1, D), lambda i: (0, 0)),
        ],
        out_specs=[
            pl.BlockSpec((blk, D), lambda i: (i, 0)),
            pl.BlockSpec((blk, D), lambda i: (i, 0)),
        ],
        out_shape=[
            jax.ShapeDtypeStruct((ROWS, D), jnp.float32),
            jax.ShapeDtypeStruct((ROWS, D), jnp.float32),
        ],
    )(p, sel, b1, wl23, bl23, wr23, br23)


def _post_body(p_ref, b2_ref, b3_ref, mu_ref, lv_ref):
    p0 = p_ref[0]
    p1 = p_ref[1]
    half = D // 2
    den2 = p0[:, D:D + 1] + p1[:, D:D + 1]
    den3 = p0[:, D + 1:D + 2] + p1[:, D + 1:D + 2]
    mu_ref[...] = (p0[:, :half] + p1[:, :half]) / (den2 + 1e-16) + b2_ref[...]
    lv_ref[...] = (p0[:, half:D] + p1[:, half:D]) / (den3 + 1e-16) + b3_ref[...]


@jax.jit
def _post(p, b2, b3):
    blk = 1024
    grid = ROWS // blk
    half = D // 2
    return pl.pallas_call(
        _post_body,
        grid=(grid,),
        in_specs=[
            pl.BlockSpec((NC, blk, ACC_W), lambda i: (0, i, 0)),
            pl.BlockSpec((1, half), lambda i: (0, 0)),
            pl.BlockSpec((1, half), lambda i: (0, 0)),
        ],
        out_specs=[
            pl.BlockSpec((blk, half), lambda i: (i, 0)),
            pl.BlockSpec((blk, half), lambda i: (i, 0)),
        ],
        out_shape=[
            jax.ShapeDtypeStruct((ROWS, half), jnp.float32),
            jax.ShapeDtypeStruct((ROWS, half), jnp.float32),
        ],
    )(p, b2, b3)


def _rot_att(att_flat, cph):
    c = jnp.arange(D)[:, None]
    l = jnp.arange(16)[None, :]
    chan = (c // cph) * cph + ((c % cph) + l) % cph
    return att_flat[chan]


def kernel(x, edge_index, Wl1, bl1, Wr1, br1, att1, b1,
           Wl2, bl2, Wr2, br2, att2, b2,
           Wl3, bl3, Wr3, br3, att3, b3):
    loop = jnp.arange(N, dtype=jnp.int32)
    pad_e = E_PAD - E_TOT
    src = jnp.concatenate(
        [edge_index[0], loop, jnp.zeros((pad_e,), jnp.int32)]).reshape(G, K)
    dst = jnp.concatenate(
        [edge_index[1], loop, jnp.full((pad_e,), N, jnp.int32)]).reshape(G, K)

    x_pad = jnp.zeros((ROWS, D_IN), jnp.float32).at[:N].set(x)

    xl1, xr1 = _pre(x_pad, Wl1, bl1.reshape(1, -1), Wr1, br1.reshape(1, -1))
    p1 = _edge_pass_h4(xl1, xr1, src, dst, _rot_att(att1.reshape(-1), 16))

    wl23 = jnp.concatenate([Wl2, Wl3], axis=1)
    bl23 = jnp.concatenate([bl2, bl3]).reshape(1, -1)
    wr23 = jnp.concatenate([Wr2, Wr3], axis=1)
    br23 = jnp.concatenate([br2, br3]).reshape(1, -1)
    a23, b23v = _mid(p1, b1.reshape(1, -1), wl23, bl23, wr23, br23)

    att23 = jnp.concatenate([att2.reshape(-1), att3.reshape(-1)])
    p2 = _edge_pass_h2(a23, b23v, src, dst, _rot_att(att23, 32))

    mu, lv = _post(p2, b2.reshape(1, -1), b3.reshape(1, -1))
    return mu[:N], lv[:N]


# speedup vs baseline: 1.2554x; 1.2554x over previous
import functools

import jax
import jax.numpy as jnp
import numpy as np
from jax import lax
from jax.experimental import pallas as pl
from jax.experimental.pallas import tpu as pltpu
from jax.experimental.pallas import tpu_sc as plsc

N = 10000
ROWS = 10240
D_IN = 128
D = 64
DW = D // 2
ACC_W = 80
E_RAW = 320000
E_TOT = E_RAW + N
K = 128
NC, NS = 2, 16
NW = NC * NS
CPW = 2 * (-(-E_TOT // (2 * K * NW)))
G = CPW * NW
E_PAD = G * K
RPT = ROWS // NS
NCOPY = RPT // K


def _edge_body(H, a_hbm, b_hbm, src_hbm, dst_hbm, att_hbm, out_hbm,
               idx_all_s, idx_all_d, arows, brows, stage, att_v, acc,
               sem_a, sem_b, sem_s):
    cid = lax.axis_index("c")
    sid = lax.axis_index("s")
    cph = D // H

    pltpu.sync_copy(att_hbm, att_v)

    zero = jnp.zeros((16,), jnp.float32)

    def zrow(i, c):
        for p in range(2):
            for j in range(ACC_W // 16):
                stage[p, i, pl.ds(j * 16, 16)] = zero
        return c

    lax.fori_loop(0, K, zrow, 0)
    for j in range(NCOPY):
        pltpu.sync_copy(stage.at[0], acc.at[pl.ds(sid * RPT + j * K, K)])
    plsc.subcore_barrier()

    wid = sid * NC + cid
    lane = lax.iota(jnp.int32, 16)

    pltpu.sync_copy(src_hbm.at[pl.ds(wid * CPW, CPW)], idx_all_s)
    pltpu.sync_copy(dst_hbm.at[pl.ds(wid * CPW, CPW)], idx_all_d)

    def gather_pair(p, t):
        pltpu.async_copy(a_hbm.at[idx_all_s.at[t]], arows.at[p], sem_a.at[p])
        pltpu.async_copy(b_hbm.at[idx_all_d.at[t]], brows.at[p], sem_b.at[p])

    wph = cph // 2

    def unpack_word(w16):
        return plsc.unpack(plsc.bitcast(w16, jnp.bfloat16),
                           format=plsc.PackFormat.INTERLEAVED,
                           preferred_element_type=jnp.float32)

    def compute_chunk(p):
        def group_body(gg, c2):
            rows = lane + gg * 16
            wcols = [(w // wph) * wph + ((w % wph + lane) % wph)
                     for w in range(DW)]
            scores = [jnp.zeros((16,), jnp.float32) for _ in range(H)]
            for w in range(DW):
                alo, ahi = unpack_word(plsc.load_gather(arows.at[p], [rows, wcols[w]]))
                blo, bhi = unpack_word(plsc.load_gather(brows.at[p], [rows, wcols[w]]))
                vlo = alo + blo
                vhi = ahi + bhi
                elo = jnp.maximum(vlo, 0.2 * vlo)
                ehi = jnp.maximum(vhi, 0.2 * vhi)
                scores[w // wph] = scores[w // wph] + att_v[2 * w] * elo + att_v[2 * w + 1] * ehi
            exs = []
            for h in range(H):
                exh = jnp.exp(scores[h])
                exs.append(exh)
                plsc.store_scatter(stage.at[p], [rows, jnp.full((16,), D + h, jnp.int32)], exh)
            for w in range(DW):
                alo, ahi = unpack_word(plsc.load_gather(arows.at[p], [rows, wcols[w]]))
                exw = exs[w // wph]
                clo = 2 * wcols[w]
                plsc.store_scatter(stage.at[p], [rows, clo], exw * alo)
                plsc.store_scatter(stage.at[p], [rows, clo + 1], exw * ahi)
            return c2

        lax.fori_loop(0, K // 16, group_body, 0)

    pltpu.async_copy(stage.at[1], acc.at[idx_all_d.at[0]], sem_s.at[1], add=True)
    gather_pair(0, 0)

    def pipe_body(t0, c):
        for b in range(2):
            t = t0 + b
            p = b
            q = 1 - b
            pltpu.make_async_copy(stage.at[q], acc.at[idx_all_d.at[t]], sem_s.at[q]).wait()

            @pl.when(t + 1 < CPW)
            def _():
                gather_pair(q, t + 1)

            pltpu.make_async_copy(a_hbm.at[idx_all_s.at[t]], arows.at[p], sem_a.at[p]).wait()
            pltpu.make_async_copy(b_hbm.at[idx_all_d.at[t]], brows.at[p], sem_b.at[p]).wait()
            compute_chunk(p)
            pltpu.async_copy(stage.at[p], acc.at[idx_all_d.at[t]], sem_s.at[p], add=True)
        return c

    lax.fori_loop(0, CPW // 2, lambda i, c: pipe_body(i * 2, c), 0)
    pltpu.make_async_copy(stage.at[1], acc.at[idx_all_d.at[0]], sem_s.at[1]).wait()
    plsc.subcore_barrier()
    for j in range(NCOPY):
        sl = pl.ds(sid * RPT + j * K, K)
        pltpu.sync_copy(acc.at[sl], out_hbm.at[cid, sl])


def _make_edge_pass(H):
    body = functools.partial(_edge_body, H)
    return pl.kernel(
        body,
        out_type=jax.ShapeDtypeStruct((NC, ROWS, ACC_W), jnp.float32),
        mesh=plsc.VectorSubcoreMesh(core_axis_name="c", subcore_axis_name="s"),
        scratch_types=[
            pltpu.VMEM((CPW, K), jnp.int32),
            pltpu.VMEM((CPW, K), jnp.int32),
            pltpu.VMEM((2, K, DW), jnp.int32),
            pltpu.VMEM((2, K, DW), jnp.int32),
            pltpu.VMEM((2, K, ACC_W), jnp.float32),
            pltpu.VMEM((2 * DW, 16), jnp.float32),
            pltpu.VMEM_SHARED((ROWS, ACC_W), jnp.float32),
            pltpu.SemaphoreType.DMA((2,)),
            pltpu.SemaphoreType.DMA((2,)),
            pltpu.SemaphoreType.DMA((2,)),
        ],
        compiler_params=pltpu.CompilerParams(needs_layout_passes=False,
                                             use_tc_tiling_on_sc=False),
    )


_edge_pass_h4 = _make_edge_pass(4)
_edge_pass_h2 = _make_edge_pass(2)


def _pre_body(x_ref, wl_ref, bl_ref, wr_ref, br_ref, xl_ref, xr_ref):
    xv = x_ref[...]
    xl_ref[...] = (jnp.dot(xv, wl_ref[...], preferred_element_type=jnp.float32)
                   + bl_ref[...]).astype(jnp.bfloat16)
    xr_ref[...] = (jnp.dot(xv, wr_ref[...], preferred_element_type=jnp.float32)
                   + br_ref[...]).astype(jnp.bfloat16)


@jax.jit
def _pre(x, wl, bl, wr, br):
    blk = 1024
    grid = ROWS // blk
    return pl.pallas_call(
        _pre_body,
        grid=(grid,),
        in_specs=[
            pl.BlockSpec((blk, D_IN), lambda i: (i, 0)),
            pl.BlockSpec((D_IN, D), lambda i: (0, 0)),
            pl.BlockSpec((1, D), lambda i: (0, 0)),
            pl.BlockSpec((D_IN, D), lambda i: (0, 0)),
            pl.BlockSpec((1, D), lambda i: (0, 0)),
        ],
        out_specs=[
            pl.BlockSpec((blk, D), lambda i: (i, 0)),
            pl.BlockSpec((blk, D), lambda i: (i, 0)),
        ],
        out_shape=[
            jax.ShapeDtypeStruct((ROWS, D), jnp.bfloat16),
            jax.ShapeDtypeStruct((ROWS, D), jnp.bfloat16),
        ],
    )(x, wl, bl, wr, br)


def _mid_body(p_ref, sel_ref, b1_ref, wl_ref, bl_ref, wr_ref, br_ref, a_ref, b_ref):
    p0 = p_ref[0]
    p1 = p_ref[1]
    num = p0[:, :D] + p1[:, :D]
    den4 = p0[:, D:D + 4] + p1[:, D:D + 4]
    den = jnp.dot(den4, sel_ref[...], preferred_element_type=jnp.float32)
    h = jnp.maximum(num / (den + 1e-16) + b1_ref[...], 0.0)
    a_ref[...] = (jnp.dot(h, wl_ref[...], preferred_element_type=jnp.float32)
                  + bl_ref[...]).astype(jnp.bfloat16)
    b_ref[...] = (jnp.dot(h, wr_ref[...], preferred_element_type=jnp.float32)
                  + br_ref[...]).astype(jnp.bfloat16)


@jax.jit
def _mid(p, b1, wl23, bl23, wr23, br23):
    blk = 1024
    grid = ROWS // blk
    sel = jnp.asarray(np.repeat(np.eye(4, dtype=np.float32), 16, axis=1))
    return pl.pallas_call(
        _mid_body,
        grid=(grid,),
        in_specs=[
            pl.BlockSpec((NC, blk, ACC_W), lambda i: (0, i, 0)),
            pl.BlockSpec((4, D), lambda i: (0, 0)),
            pl.BlockSpec((1, D), lambda i: (0, 0)),
            pl.BlockSpec((D, D), lambda i: (0, 0)),
            pl.BlockSpec((1, D), lambda i: (0, 0)),
            pl.BlockSpec((D, D), lambda i: (0, 0)),
            pl.BlockSpec((1, D), lambda i: (0, 0)),
        ],
        out_specs=[
            pl.BlockSpec((blk, D), lambda i: (i, 0)),
            pl.BlockSpec((blk, D), lambda i: (i, 0)),
        ],
        out_shape=[
            jax.ShapeDtypeStruct((ROWS, D), jnp.bfloat16),
            jax.ShapeDtypeStruct((ROWS, D), jnp.bfloat16),
        ],
    )(p, sel, b1, wl23, bl23, wr23, br23)


def _post_body(p_ref, b2_ref, b3_ref, mu_ref, lv_ref):
    p0 = p_ref[0]
    p1 = p_ref[1]
    half = D // 2
    den2 = p0[:, D:D + 1] + p1[:, D:D + 1]
    den3 = p0[:, D + 1:D + 2] + p1[:, D + 1:D + 2]
    mu_ref[...] = (p0[:, :half] + p1[:, :half]) / (den2 + 1e-16) + b2_ref[...]
    lv_ref[...] = (p0[:, half:D] + p1[:, half:D]) / (den3 + 1e-16) + b3_ref[...]


@jax.jit
def _post(p, b2, b3):
    blk = 1024
    grid = ROWS // blk
    half = D // 2
    return pl.pallas_call(
        _post_body,
        grid=(grid,),
        in_specs=[
            pl.BlockSpec((NC, blk, ACC_W), lambda i: (0, i, 0)),
            pl.BlockSpec((1, half), lambda i: (0, 0)),
            pl.BlockSpec((1, half), lambda i: (0, 0)),
        ],
        out_specs=[
            pl.BlockSpec((blk, half), lambda i: (i, 0)),
            pl.BlockSpec((blk, half), lambda i: (i, 0)),
        ],
        out_shape=[
            jax.ShapeDtypeStruct((ROWS, half), jnp.float32),
            jax.ShapeDtypeStruct((ROWS, half), jnp.float32),
        ],
    )(p, b2, b3)


def _pack_i32(t):
    return jax.lax.bitcast_convert_type(t.reshape(ROWS, DW, 2), jnp.int32)


def _rot_att(att_flat, cph):
    wph = cph // 2
    w = jnp.arange(DW)[:, None]
    l = jnp.arange(16)[None, :]
    wchan = (w // wph) * wph + ((w % wph) + l) % wph
    lo = att_flat[2 * wchan]
    hi = att_flat[2 * wchan + 1]
    return jnp.stack([lo, hi], axis=1).reshape(2 * DW, 16)


def kernel(x, edge_index, Wl1, bl1, Wr1, br1, att1, b1,
           Wl2, bl2, Wr2, br2, att2, b2,
           Wl3, bl3, Wr3, br3, att3, b3):
    loop = jnp.arange(N, dtype=jnp.int32)
    pad_e = E_PAD - E_TOT
    src = jnp.concatenate(
        [edge_index[0], loop, jnp.zeros((pad_e,), jnp.int32)]).reshape(G, K)
    dst = jnp.concatenate(
        [edge_index[1], loop, jnp.full((pad_e,), N, jnp.int32)]).reshape(G, K)

    x_pad = jnp.zeros((ROWS, D_IN), jnp.float32).at[:N].set(x)

    xl1, xr1 = _pre(x_pad, Wl1, bl1.reshape(1, -1), Wr1, br1.reshape(1, -1))
    p1 = _edge_pass_h4(_pack_i32(xl1), _pack_i32(xr1), src, dst,
                       _rot_att(att1.reshape(-1), 16))

    wl23 = jnp.concatenate([Wl2, Wl3], axis=1)
    bl23 = jnp.concatenate([bl2, bl3]).reshape(1, -1)
    wr23 = jnp.concatenate([Wr2, Wr3], axis=1)
    br23 = jnp.concatenate([br2, br3]).reshape(1, -1)
    a23, b23v = _mid(p1, b1.reshape(1, -1), wl23, bl23, wr23, br23)

    att23 = jnp.concatenate([att2.reshape(-1), att3.reshape(-1)])
    p2 = _edge_pass_h2(_pack_i32(a23), _pack_i32(b23v), src, dst,
                       _rot_att(att23, 32))

    mu, lv = _post(p2, b2.reshape(1, -1), b3.reshape(1, -1))
    return mu[:N], lv[:N]


# speedup vs baseline: 1.3164x; 1.0486x over previous
import functools

import jax
import jax.numpy as jnp
import numpy as np
from jax import lax
from jax.experimental import pallas as pl
from jax.experimental.pallas import tpu as pltpu
from jax.experimental.pallas import tpu_sc as plsc

N = 10000
ROWS = 10240
D_IN = 128
D = 64
DW = D // 2
ACC_W = 80
E_RAW = 320000
E_TOT = E_RAW + N
K = 128
NC, NS = 2, 16
NW = NC * NS
CPW = 2 * (-(-E_TOT // (2 * K * NW)))
G = CPW * NW
E_PAD = G * K
RPT = ROWS // NS
NCOPY = RPT // K


def _edge_body(H, a_hbm, b_hbm, src_hbm, dst_hbm, att_hbm, out_hbm,
               idx_all_s, idx_all_d, arows, brows, stage, att_v, acc,
               sem_a, sem_b, sem_s):
    cid = lax.axis_index("c")
    sid = lax.axis_index("s")
    cph = D // H

    pltpu.sync_copy(att_hbm, att_v)

    zero = jnp.zeros((16,), jnp.float32)

    def zrow(i, c):
        for p in range(2):
            for j in range(ACC_W // 16):
                stage[p, i, pl.ds(j * 16, 16)] = zero
        return c

    lax.fori_loop(0, K, zrow, 0)
    for j in range(NCOPY):
        pltpu.sync_copy(stage.at[0], acc.at[pl.ds(sid * RPT + j * K, K)])
    plsc.subcore_barrier()

    wid = sid * NC + cid
    lane = lax.iota(jnp.int32, 16)

    pltpu.sync_copy(src_hbm.at[pl.ds(wid * CPW, CPW)], idx_all_s)
    pltpu.sync_copy(dst_hbm.at[pl.ds(wid * CPW, CPW)], idx_all_d)

    def gather_pair(p, t):
        pltpu.async_copy(a_hbm.at[idx_all_s.at[t]], arows.at[p], sem_a.at[p])
        pltpu.async_copy(b_hbm.at[idx_all_d.at[t]], brows.at[p], sem_b.at[p])

    wph = cph // 2

    def unpack_word(w16):
        return plsc.unpack(plsc.bitcast(w16, jnp.bfloat16),
                           format=plsc.PackFormat.INTERLEAVED,
                           preferred_element_type=jnp.float32)

    def compute_chunk(p):
        def group_body(gg, c2):
            rows = lane + gg * 16
            wcols = [(w // wph) * wph + ((w % wph + lane) % wph)
                     for w in range(DW)]
            scores = [jnp.zeros((16,), jnp.float32) for _ in range(H)]
            for w in range(DW):
                abf = plsc.bitcast(plsc.load_gather(arows.at[p], [rows, wcols[w]]), jnp.bfloat16)
                bbf = plsc.bitcast(plsc.load_gather(brows.at[p], [rows, wcols[w]]), jnp.bfloat16)
                v = abf + bbf
                e = jnp.maximum(v, jnp.bfloat16(0.2) * v)
                elo, ehi = plsc.unpack(e, format=plsc.PackFormat.INTERLEAVED,
                                       preferred_element_type=jnp.float32)
                scores[w // wph] = scores[w // wph] + att_v[2 * w] * elo + att_v[2 * w + 1] * ehi
            exs = []
            for h in range(H):
                exh = jnp.exp(scores[h])
                exs.append(exh)
                plsc.store_scatter(stage.at[p], [rows, jnp.full((16,), D + h, jnp.int32)], exh)
            for w in range(DW):
                alo, ahi = unpack_word(plsc.load_gather(arows.at[p], [rows, wcols[w]]))
                exw = exs[w // wph]
                clo = 2 * wcols[w]
                plsc.store_scatter(stage.at[p], [rows, clo], exw * alo)
                plsc.store_scatter(stage.at[p], [rows, clo + 1], exw * ahi)
            return c2

        lax.fori_loop(0, K // 16, group_body, 0)

    pltpu.async_copy(stage.at[1], acc.at[idx_all_d.at[0]], sem_s.at[1], add=True)
    gather_pair(0, 0)

    def pipe_body(t0, c):
        for b in range(2):
            t = t0 + b
            p = b
            q = 1 - b
            pltpu.make_async_copy(stage.at[q], acc.at[idx_all_d.at[t]], sem_s.at[q]).wait()

            @pl.when(t + 1 < CPW)
            def _():
                gather_pair(q, t + 1)

            pltpu.make_async_copy(a_hbm.at[idx_all_s.at[t]], arows.at[p], sem_a.at[p]).wait()
            pltpu.make_async_copy(b_hbm.at[idx_all_d.at[t]], brows.at[p], sem_b.at[p]).wait()
            compute_chunk(p)
            pltpu.async_copy(stage.at[p], acc.at[idx_all_d.at[t]], sem_s.at[p], add=True)
        return c

    lax.fori_loop(0, CPW // 2, lambda i, c: pipe_body(i * 2, c), 0)
    pltpu.make_async_copy(stage.at[1], acc.at[idx_all_d.at[0]], sem_s.at[1]).wait()
    plsc.subcore_barrier()
    for j in range(NCOPY):
        sl = pl.ds(sid * RPT + j * K, K)
        pltpu.sync_copy(acc.at[sl], out_hbm.at[cid, sl])


def _make_edge_pass(H):
    body = functools.partial(_edge_body, H)
    return pl.kernel(
        body,
        out_type=jax.ShapeDtypeStruct((NC, ROWS, ACC_W), jnp.float32),
        mesh=plsc.VectorSubcoreMesh(core_axis_name="c", subcore_axis_name="s"),
        scratch_types=[
            pltpu.VMEM((CPW, K), jnp.int32),
            pltpu.VMEM((CPW, K), jnp.int32),
            pltpu.VMEM((2, K, DW), jnp.int32),
            pltpu.VMEM((2, K, DW), jnp.int32),
            pltpu.VMEM((2, K, ACC_W), jnp.float32),
            pltpu.VMEM((2 * DW, 16), jnp.float32),
            pltpu.VMEM_SHARED((ROWS, ACC_W), jnp.float32),
            pltpu.SemaphoreType.DMA((2,)),
            pltpu.SemaphoreType.DMA((2,)),
            pltpu.SemaphoreType.DMA((2,)),
        ],
        compiler_params=pltpu.CompilerParams(needs_layout_passes=False,
                                             use_tc_tiling_on_sc=False),
    )


_edge_pass_h4 = _make_edge_pass(4)
_edge_pass_h2 = _make_edge_pass(2)


def _pre_body(x_ref, wl_ref, bl_ref, wr_ref, br_ref, xl_ref, xr_ref):
    xv = x_ref[...]
    xl_ref[...] = (jnp.dot(xv, wl_ref[...], preferred_element_type=jnp.float32)
                   + bl_ref[...]).astype(jnp.bfloat16)
    xr_ref[...] = (jnp.dot(xv, wr_ref[...], preferred_element_type=jnp.float32)
                   + br_ref[...]).astype(jnp.bfloat16)


@jax.jit
def _pre(x, wl, bl, wr, br):
    blk = 1024
    grid = ROWS // blk
    return pl.pallas_call(
        _pre_body,
        grid=(grid,),
        in_specs=[
            pl.BlockSpec((blk, D_IN), lambda i: (i, 0)),
            pl.BlockSpec((D_IN, D), lambda i: (0, 0)),
            pl.BlockSpec((1, D), lambda i: (0, 0)),
            pl.BlockSpec((D_IN, D), lambda i: (0, 0)),
            pl.BlockSpec((1, D), lambda i: (0, 0)),
        ],
        out_specs=[
            pl.BlockSpec((blk, D), lambda i: (i, 0)),
            pl.BlockSpec((blk, D), lambda i: (i, 0)),
        ],
        out_shape=[
            jax.ShapeDtypeStruct((ROWS, D), jnp.bfloat16),
            jax.ShapeDtypeStruct((ROWS, D), jnp.bfloat16),
        ],
    )(x, wl, bl, wr, br)


def _mid_body(p_ref, sel_ref, b1_ref, wl_ref, bl_ref, wr_ref, br_ref, a_ref, b_ref):
    p0 = p_ref[0]
    p1 = p_ref[1]
    num = p0[:, :D] + p1[:, :D]
    den4 = p0[:, D:D + 4] + p1[:, D:D + 4]
    den = jnp.dot(den4, sel_ref[...], preferred_element_type=jnp.float32)
    h = jnp.maximum(num / (den + 1e-16) + b1_ref[...], 0.0)
    a_ref[...] = (jnp.dot(h, wl_ref[...], preferred_element_type=jnp.float32)
                  + bl_ref[...]).astype(jnp.bfloat16)
    b_ref[...] = (jnp.dot(h, wr_ref[...], preferred_element_type=jnp.float32)
                  + br_ref[...]).astype(jnp.bfloat16)


@jax.jit
def _mid(p, b1, wl23, bl23, wr23, br23):
    blk = 1024
    grid = ROWS // blk
    sel = jnp.asarray(np.repeat(np.eye(4, dtype=np.float32), 16, axis=1))
    return pl.pallas_call(
        _mid_body,
        grid=(grid,),
        in_specs=[
            pl.BlockSpec((NC, blk, ACC_W), lambda i: (0, i, 0)),
            pl.BlockSpec((4, D), lambda i: (0, 0)),
            pl.BlockSpec((1, D), lambda i: (0, 0)),
            pl.BlockSpec((D, D), lambda i: (0, 0)),
            pl.BlockSpec((1, D), lambda i: (0, 0)),
            pl.BlockSpec((D, D), lambda i: (0, 0)),
            pl.BlockSpec((1, D), lambda i: (0, 0)),
        ],
        out_specs=[
            pl.BlockSpec((blk, D), lambda i: (i, 0)),
            pl.BlockSpec((blk, D), lambda i: (i, 0)),
        ],
        out_shape=[
            jax.ShapeDtypeStruct((ROWS, D), jnp.bfloat16),
            jax.ShapeDtypeStruct((ROWS, D), jnp.bfloat16),
        ],
    )(p, sel, b1, wl23, bl23, wr23, br23)


def _post_body(p_ref, b2_ref, b3_ref, mu_ref, lv_ref):
    p0 = p_ref[0]
    p1 = p_ref[1]
    half = D // 2
    den2 = p0[:, D:D + 1] + p1[:, D:D + 1]
    den3 = p0[:, D + 1:D + 2] + p1[:, D + 1:D + 2]
    mu_ref[...] = (p0[:, :half] + p1[:, :half]) / (den2 + 1e-16) + b2_ref[...]
    lv_ref[...] = (p0[:, half:D] + p1[:, half:D]) / (den3 + 1e-16) + b3_ref[...]


@jax.jit
def _post(p, b2, b3):
    blk = 1024
    grid = ROWS // blk
    half = D // 2
    return pl.pallas_call(
        _post_body,
        grid=(grid,),
        in_specs=[
            pl.BlockSpec((NC, blk, ACC_W), lambda i: (0, i, 0)),
            pl.BlockSpec((1, half), lambda i: (0, 0)),
            pl.BlockSpec((1, half), lambda i: (0, 0)),
        ],
        out_specs=[
            pl.BlockSpec((blk, half), lambda i: (i, 0)),
            pl.BlockSpec((blk, half), lambda i: (i, 0)),
        ],
        out_shape=[
            jax.ShapeDtypeStruct((ROWS, half), jnp.float32),
            jax.ShapeDtypeStruct((ROWS, half), jnp.float32),
        ],
    )(p, b2, b3)


def _pack_i32(t):
    return jax.lax.bitcast_convert_type(t.reshape(ROWS, DW, 2), jnp.int32)


def _rot_att(att_flat, cph):
    wph = cph // 2
    w = jnp.arange(DW)[:, None]
    l = jnp.arange(16)[None, :]
    wchan = (w // wph) * wph + ((w % wph) + l) % wph
    lo = att_flat[2 * wchan]
    hi = att_flat[2 * wchan + 1]
    return jnp.stack([lo, hi], axis=1).reshape(2 * DW, 16)


def kernel(x, edge_index, Wl1, bl1, Wr1, br1, att1, b1,
           Wl2, bl2, Wr2, br2, att2, b2,
           Wl3, bl3, Wr3, br3, att3, b3):
    loop = jnp.arange(N, dtype=jnp.int32)
    pad_e = E_PAD - E_TOT
    src = jnp.concatenate(
        [edge_index[0], loop, jnp.zeros((pad_e,), jnp.int32)]).reshape(G, K)
    dst = jnp.concatenate(
        [edge_index[1], loop, jnp.full((pad_e,), N, jnp.int32)]).reshape(G, K)

    x_pad = jnp.zeros((ROWS, D_IN), jnp.float32).at[:N].set(x)

    xl1, xr1 = _pre(x_pad, Wl1, bl1.reshape(1, -1), Wr1, br1.reshape(1, -1))
    p1 = _edge_pass_h4(_pack_i32(xl1), _pack_i32(xr1), src, dst,
                       _rot_att(att1.reshape(-1), 16))

    wl23 = jnp.concatenate([Wl2, Wl3], axis=1)
    bl23 = jnp.concatenate([bl2, bl3]).reshape(1, -1)
    wr23 = jnp.concatenate([Wr2, Wr3], axis=1)
    br23 = jnp.concatenate([br2, br3]).reshape(1, -1)
    a23, b23v = _mid(p1, b1.reshape(1, -1), wl23, bl23, wr23, br23)

    att23 = jnp.concatenate([att2.reshape(-1), att3.reshape(-1)])
    p2 = _edge_pass_h2(_pack_i32(a23), _pack_i32(b23v), src, dst,
                       _rot_att(att23, 32))

    mu, lv = _post(p2, b2.reshape(1, -1), b3.reshape(1, -1))
    return mu[:N], lv[:N]
